# baseline (speedup 1.0000x reference)
import functools

import jax
import jax.numpy as jnp
from jax import lax
from jax.experimental import pallas as pl
from jax.experimental.pallas import tpu as pltpu
from jax.experimental.pallas import tpu_sc as plsc

N = 100000
D = 8
NC = 2
NS = 16
NW = NC * NS
ROWS_PER_TILE = 6272
NP = NS * ROWS_PER_TILE
SUB = 128
NSUB = ROWS_PER_TILE // SUB
LAST_NSUB = 46
LAST_TAIL = N - 15 * ROWS_PER_TILE - LAST_NSUB * SUB
E = 3_200_000
CH_E = 512
NCK = E // CH_E
NBIG = NCK - 195 * NW
NSLOT = 3
WOUT = 125

_mesh = plsc.VectorSubcoreMesh(core_axis_name="c", subcore_axis_name="s")


def _build_identity(id_ref, base):
    lanes = lax.iota(jnp.int32, 16)

    def body(i, carry):
        for k in range(SUB // 16):
            id_ref[i, pl.ds(k * 16, 16)] = base + i * SUB + k * 16 + lanes
        return carry

    lax.fori_loop(0, NSUB, body, 0)


def _load_v_into_shared(v_hbm, buf_a, shared_v, base, s, csem, lsem):
    nsub = jnp.where(s == NS - 1, LAST_NSUB, NSUB)

    def load(slot, i):
        r0 = base + i * SUB
        return pltpu.make_async_copy(v_hbm.at[pl.ds(r0, SUB)], buf_a.at[slot], csem)

    load(0, 0).start()
    load(1, 1).start()

    def body(i, carry):
        cur = lax.rem(i, NSLOT)
        pf = lax.rem(i + 2, NSLOT)
        load(cur, i).wait()
        load(pf, lax.rem(i + 2, nsub)).start()
        ld = pltpu.make_async_copy(
            buf_a.at[cur], shared_v.at[pl.ds(base + i * SUB, SUB)], lsem)
        ld.start()
        ld.wait()
        return carry

    lax.fori_loop(0, nsub, body, 0)
    load(0, 0).wait()
    load(0, 0).wait()

    @pl.when(s == NS - 1)
    def _():
        r0 = base + LAST_NSUB * SUB
        t = pltpu.make_async_copy(
            v_hbm.at[pl.ds(r0, LAST_TAIL)], buf_a.at[0, pl.ds(0, LAST_TAIL)], csem)
        t.start()
        t.wait()
        t2 = pltpu.make_async_copy(
            buf_a.at[0, pl.ds(0, LAST_TAIL)], shared_v.at[pl.ds(r0, LAST_TAIL)], lsem)
        t2.start()
        t2.wait()


def _combine_into_shared(parts, buf_a, buf_b, id_ref, shared_v, base,
                         csem, lsem, ssem):
    def loads(slot, i):
        r0 = base + i * SUB
        return (
            pltpu.make_async_copy(parts.at[0, pl.ds(r0, SUB)], buf_a.at[slot], csem),
            pltpu.make_async_copy(parts.at[1, pl.ds(r0, SUB)], buf_b.at[slot], csem),
        )

    def scat(slot, i):
        return pltpu.make_async_copy(buf_b.at[slot], shared_v.at[id_ref.at[i]], ssem)

    for d in loads(0, 0) + loads(1, 1):
        d.start()

    def body(i, carry):
        cur = lax.rem(i, NSLOT)
        pf = lax.rem(i + 2, NSLOT)
        for d in loads(cur, i):
            d.wait()
        @pl.when(i >= 1)
        def _():
            scat(pf, i - 1).wait()
        for d in loads(pf, lax.rem(i + 2, NSUB)):
            d.start()
        ld = pltpu.make_async_copy(
            buf_a.at[cur], shared_v.at[pl.ds(base + i * SUB, SUB)], lsem)
        ld.start()
        ld.wait()
        scat(cur, i).start(add=True)
        return carry

    lax.fori_loop(0, NSUB, body, 0)

    scat((NSUB - 1) % NSLOT, 0).wait()
    for slot in (NSUB % NSLOT, (NSUB + 1) % NSLOT):
        for d in loads(slot, 0):
            d.wait()


def _zero_shared_out(zeros_hbm, buf, shared_out, base, zsem):
    pltpu.sync_copy(zeros_hbm, buf)

    def issue(i, carry):
        pltpu.make_async_copy(
            buf, shared_out.at[pl.ds(base + i * SUB, SUB)], zsem).start()
        return carry

    def drain(i, carry):
        pltpu.make_async_copy(
            buf, shared_out.at[pl.ds(base, SUB)], zsem).wait()
        return carry

    lax.fori_loop(0, NSUB, issue, 0)
    lax.fori_loop(0, NSUB, drain, 0)


def _edge_phase(edges_hbm, ibuf, rows, isem, gsem, ssem,
                shared_v, shared_out, w):
    big = w < NBIG
    g0 = jnp.where(big, w * 196, NBIG * 196 + (w - NBIG) * 195)
    nck = jnp.where(big, 196, 195)

    def idx_copies(slot, row):
        e0 = row * CH_E
        return (
            pltpu.make_async_copy(
                edges_hbm.at[pl.ds(E + e0, CH_E)], ibuf.at[slot, 0], isem),
            pltpu.make_async_copy(
                edges_hbm.at[pl.ds(e0, CH_E)], ibuf.at[slot, 1], isem),
        )

    def scatter_copy(slot):
        return pltpu.make_async_copy(
            rows.at[slot], shared_out.at[ibuf.at[slot, 1]], ssem)

    for d in idx_copies(0, g0):
        d.start()

    def chunk(g, carry):
        c = lax.rem(g, NSLOT)
        c1 = lax.rem(g + 1, NSLOT)
        for d in idx_copies(c, g0 + g):
            d.wait()
        @pl.when(g >= 2)
        def _():
            scatter_copy(c1).wait()
        row_pf = jnp.where(g + 1 < nck, g0 + g + 1, g0)
        for d in idx_copies(c1, row_pf):
            d.start()
        gd = pltpu.make_async_copy(shared_v.at[ibuf.at[c, 0]], rows.at[c], gsem)
        gd.start()
        gd.wait()
        scatter_copy(c).start(add=True)
        return carry

    lax.fori_loop(0, nck, chunk, 0)

    scatter_copy(0).wait()
    scatter_copy(0).wait()
    for d in idx_copies(0, g0):
        d.wait()


def _writeout_parts(parts_out, buf, shared_out, base, c, wsem):
    def wo(slot, i):
        r0 = base + i * SUB
        return pltpu.make_async_copy(
            buf.at[slot], parts_out.at[c, pl.ds(r0, SUB)], wsem)

    def body(i, carry):
        slot = lax.rem(i, NSLOT)
        @pl.when(i >= NSLOT)
        def _():
            wo(slot, 0).wait()
        pltpu.sync_copy(shared_out.at[pl.ds(base + i * SUB, SUB)], buf.at[slot])
        wo(slot, i).start()
        return carry

    lax.fori_loop(0, NSUB, body, 0)
    for _ in range(NSLOT):
        wo(0, 0).wait()


@functools.partial(
    pl.kernel,
    out_type=jax.ShapeDtypeStruct((N, D), jnp.float32),
    mesh=_mesh,
    compiler_params=pltpu.CompilerParams(use_tc_tiling_on_sc=False),
    scratch_types=[
        pltpu.HBM((2, NC, NP, D), jnp.float32),
        pltpu.VMEM_SHARED((NP, D), jnp.float32),
        pltpu.VMEM_SHARED((NP, D), jnp.float32),
        pltpu.VMEM((NSLOT, SUB, D), jnp.float32),
        pltpu.VMEM((NSLOT, SUB, D), jnp.float32),
        pltpu.VMEM((NSUB, SUB), jnp.int32),
        pltpu.VMEM((NSLOT, 2, CH_E), jnp.int32),
        pltpu.VMEM((NSLOT, CH_E, D), jnp.float32),
        pltpu.VMEM((WOUT, D), jnp.float32),
        pltpu.SemaphoreType.DMA,
        pltpu.SemaphoreType.DMA,
        pltpu.SemaphoreType.DMA,
        pltpu.SemaphoreType.REGULAR,
    ],
)
def _fused(v_hbm, edges_hbm, zeros_hbm, v_out, parts,
           shared_v, shared_out, buf_a, buf_b, id_ref, ibuf, rows, wbuf,
           isem, gsem, ssem, xsem):
    c = lax.axis_index("c")
    s = lax.axis_index("s")
    base = s * ROWS_PER_TILE
    w = c * NS + s

    def global_barrier():
        plsc.subcore_barrier()
        pl.semaphore_signal(xsem, 1, core_index=1 - c)
        pl.semaphore_wait(xsem, 1)

    _build_identity(id_ref, base)

    _load_v_into_shared(v_hbm, buf_a, shared_v, base, s, isem, gsem)
    _zero_shared_out(zeros_hbm, buf_a.at[0], shared_out, base, ssem)
    plsc.subcore_barrier()
    _edge_phase(edges_hbm, ibuf, rows, isem, gsem, ssem, shared_v, shared_out, w)
    plsc.subcore_barrier()
    _writeout_parts(parts.at[0], buf_a, shared_out, base, c, gsem)

    for it in (1, 2):
        global_barrier()
        _combine_into_shared(parts.at[(it - 1) % 2], buf_a, buf_b, id_ref,
                             shared_v, base, isem, gsem, ssem)
        _zero_shared_out(zeros_hbm, buf_a.at[0], shared_out, base, ssem)
        plsc.subcore_barrier()
        _edge_phase(edges_hbm, ibuf, rows, isem, gsem, ssem,
                    shared_v, shared_out, w)
        plsc.subcore_barrier()
        _writeout_parts(parts.at[it % 2], buf_a, shared_out, base, c, gsem)

    global_barrier()
    _combine_into_shared(parts.at[0], buf_a, buf_b, id_ref, shared_v, base,
                         isem, gsem, ssem)
    plsc.subcore_barrier()
    o0 = w * (N // NW)
    for i in range(N // NW // WOUT):
        r0 = o0 + i * WOUT
        pltpu.sync_copy(shared_v.at[pl.ds(r0, WOUT)], wbuf)
        pltpu.sync_copy(wbuf, v_out.at[pl.ds(r0, WOUT)])


def kernel(v0, edge_index):
    edges = edge_index.reshape(-1).astype(jnp.int32)
    zeros = jnp.zeros((SUB, D), jnp.float32)
    return _fused(v0, edges, zeros)
